# baseline (speedup 1.0000x reference)
import functools

import jax
import jax.numpy as jnp
from jax import lax
from jax.experimental import pallas as pl
from jax.experimental.pallas import tpu as pltpu
from jax.experimental.pallas import tpu_sc as plsc

N_NODES = 50000
D = 128
E = 200000
NREL = 3

NC = 2
NS = 16
FC = 32
NFC = D // FC
PASSES = NFC // NC
UN = 256
UNITS = 50
EPAD = NS * UNITS * UN

ACC_ROWS = 50176
ZROWS = ACC_ROWS // NS
ZCH = 112
K = 5
NG = UNITS // K
DUMMY = N_NODES


def _mm_body(x_ref, w_ref, o_ref):
    for r in range(NREL):
        o_ref[r] = jnp.dot(x_ref[...], w_ref[r],
                           preferred_element_type=jnp.float32
                           ).astype(jnp.bfloat16)


def _matmul(x, w_stack):
    mblk = 400
    return pl.pallas_call(
        _mm_body,
        grid=(N_NODES // mblk,),
        in_specs=[pl.BlockSpec((mblk, D), lambda i: (i, 0)),
                  pl.BlockSpec((NREL, D, D), lambda i: (0, 0, 0))],
        out_specs=pl.BlockSpec((NREL, mblk, D), lambda i: (0, i, 0)),
        out_shape=jax.ShapeDtypeStruct((NREL, N_NODES, D), jnp.bfloat16),
    )(x, w_stack)


_MESH = plsc.VectorSubcoreMesh(core_axis_name="c", subcore_axis_name="s")


@functools.partial(
    pl.kernel,
    out_type=jax.ShapeDtypeStruct((NFC, ACC_ROWS, FC), jnp.bfloat16),
    mesh=_MESH,
    scratch_types=[
        pltpu.VMEM_SHARED((ACC_ROWS, FC), jnp.bfloat16),
        pltpu.VMEM((UNITS, UN), jnp.int32),
        pltpu.VMEM((UNITS, UN), jnp.int32),
        pltpu.VMEM((K, UN, FC), jnp.bfloat16),
        pltpu.VMEM((ZCH, FC), jnp.bfloat16),
        pltpu.VMEM((ZCH, FC), jnp.bfloat16),
        pltpu.SemaphoreType.DMA((K,)),
        pltpu.SemaphoreType.DMA((K,)),
    ],
    compiler_params=pltpu.CompilerParams(use_tc_tiling_on_sc=False),
)
def _sc_agg(h_hbm, src0, dst0, src1, dst1, src2, dst2, out_hbm,
            acc, srcbuf, dstbuf, gbuf, zbuf, dbuf, semg, sems):
    core = lax.axis_index("c")
    tid = lax.axis_index("s")

    def _zb(i, c):
        zbuf[i, pl.ds(0, FC)] = jnp.zeros((FC,), jnp.bfloat16)
        return c
    lax.fori_loop(0, ZCH, _zb, 0)

    for p in range(PASSES):
        fc = core * PASSES + p

        for z in range(ZROWS // ZCH):
            pltpu.sync_copy(zbuf, acc.at[pl.ds(tid * ZROWS + z * ZCH, ZCH)])
        plsc.subcore_barrier()

        for r, (sh, dh) in enumerate(((src0, dst0), (src1, dst1),
                                      (src2, dst2))):
            off = r * (NFC * N_NODES) + fc
            pltpu.sync_copy(sh.at[tid], srcbuf)
            pltpu.sync_copy(dh.at[tid], dstbuf)

            def _off(j, c):
                for t in range(UN // 16):
                    v = srcbuf[j, pl.ds(t * 16, 16)]
                    srcbuf[j, pl.ds(t * 16, 16)] = v * NFC + off
                return c
            lax.fori_loop(0, UNITS, _off, 0)

            for b in range(K):
                pltpu.async_copy(h_hbm.at[srcbuf.at[b]],
                                 gbuf.at[b], semg.at[b])

            def _grp(g, c):
                for b in range(K):
                    pltpu.make_async_copy(h_hbm.at[srcbuf.at[0]],
                                          gbuf.at[b], semg.at[b]).wait()
                    pltpu.async_copy(gbuf.at[b],
                                     acc.at[dstbuf.at[g * K + b]],
                                     sems.at[b], add=True)

                @pl.when(g < NG - 1)
                def _next():
                    for b in range(K):
                        pltpu.make_async_copy(gbuf.at[b], acc.at[dstbuf.at[0]],
                                              sems.at[b]).wait()
                        pltpu.async_copy(h_hbm.at[srcbuf.at[(g + 1) * K + b]],
                                         gbuf.at[b], semg.at[b])
                return c
            lax.fori_loop(0, NG, _grp, 0)
            for b in range(K):
                pltpu.make_async_copy(gbuf.at[b], acc.at[dstbuf.at[0]],
                                      sems.at[b]).wait()

        plsc.subcore_barrier()

        for dch in range(ZROWS // ZCH):
            row0 = tid * ZROWS + dch * ZCH
            pltpu.sync_copy(acc.at[pl.ds(row0, ZCH)], dbuf)

            def _relu(i, c):
                v = dbuf[i, pl.ds(0, FC)]
                dbuf[i, pl.ds(0, FC)] = jnp.maximum(
                    v, jnp.zeros((FC,), jnp.bfloat16))
                return c
            lax.fori_loop(0, ZCH, _relu, 0)
            pltpu.sync_copy(dbuf, out_hbm.at[fc, pl.ds(row0, ZCH)])
        plsc.subcore_barrier()


def kernel(x, edge_index_r0, edge_index_r1, edge_index_r2,
           W_r0, W_r1, W_r2):
    h = _matmul(x, jnp.stack((W_r0, W_r1, W_r2)))
    h_all = h.reshape(NREL * N_NODES * NFC, FC)
    args = [h_all]
    for ei in (edge_index_r0, edge_index_r1, edge_index_r2):
        src = jnp.concatenate((ei[0], jnp.zeros((EPAD - E,), jnp.int32)))
        dst = jnp.concatenate((ei[1], jnp.full((EPAD - E,), DUMMY,
                                               jnp.int32)))
        args.append(src.reshape(NS, UNITS, UN))
        args.append(dst.reshape(NS, UNITS, UN))
    out4 = _sc_agg(*args)
    return (out4[:, :N_NODES, :].transpose(1, 0, 2)
            .reshape(N_NODES, D).astype(jnp.float32))


# speedup vs baseline: 1.2738x; 1.2738x over previous
import functools

import jax
import jax.numpy as jnp
from jax import lax
from jax.experimental import pallas as pl
from jax.experimental.pallas import tpu as pltpu
from jax.experimental.pallas import tpu_sc as plsc

N_NODES = 50000
D = 128
E = 200000
NREL = 3

NC = 2
NS = 16
FC = 32
NFC = D // FC
PASSES = NFC // NC
UN = 256
UNITS = 50
EPAD = NS * UNITS * UN

ACC_ROWS = 50176
ZROWS = ACC_ROWS // NS
ZCH = 112
K = 5
NG = UNITS // K
DUMMY = N_NODES
DROWS = N_NODES // NS
DCH = 125

_PERM = []
for _c in range(NFC):
    for _k in range(FC // 2):
        _PERM += [_c * FC + _k, _c * FC + FC // 2 + _k]


def _mm_body(x_ref, w_ref, o_ref):
    for r in range(NREL):
        o_ref[r] = jnp.dot(x_ref[...], w_ref[r],
                           preferred_element_type=jnp.float32
                           ).astype(jnp.bfloat16)


def _matmul(x, w_stack):
    mblk = 400
    return pl.pallas_call(
        _mm_body,
        grid=(N_NODES // mblk,),
        in_specs=[pl.BlockSpec((mblk, D), lambda i: (i, 0)),
                  pl.BlockSpec((NREL, D, D), lambda i: (0, 0, 0))],
        out_specs=pl.BlockSpec((NREL, mblk, D), lambda i: (0, i, 0)),
        out_shape=jax.ShapeDtypeStruct((NREL, N_NODES, D), jnp.bfloat16),
    )(x, w_stack)


_MESH = plsc.VectorSubcoreMesh(core_axis_name="c", subcore_axis_name="s")


@functools.partial(
    pl.kernel,
    out_type=jax.ShapeDtypeStruct((N_NODES, D), jnp.float32),
    mesh=_MESH,
    scratch_types=[
        pltpu.VMEM_SHARED((ACC_ROWS, FC), jnp.bfloat16),
        pltpu.VMEM((UNITS, UN), jnp.int32),
        pltpu.VMEM((UNITS, UN), jnp.int32),
        pltpu.VMEM((K, UN, FC), jnp.bfloat16),
        pltpu.VMEM((ZCH, FC), jnp.bfloat16),
        pltpu.VMEM((DCH, FC), jnp.bfloat16),
        pltpu.VMEM((DCH, FC), jnp.float32),
        pltpu.SemaphoreType.DMA((K,)),
        pltpu.SemaphoreType.DMA((K,)),
    ],
    compiler_params=pltpu.CompilerParams(use_tc_tiling_on_sc=False,
                                         needs_layout_passes=False),
)
def _sc_agg(h_hbm, src0, dst0, src1, dst1, src2, dst2, out_hbm,
            acc, srcbuf, dstbuf, gbuf, zbuf, dbuf, dbuf32, semg, sems):
    core = lax.axis_index("c")
    tid = lax.axis_index("s")

    def _zb(i, c):
        zbuf[i, pl.ds(0, FC)] = jnp.zeros((FC,), jnp.bfloat16)
        return c
    lax.fori_loop(0, ZCH, _zb, 0)

    for p in range(PASSES):
        fc = core * PASSES + p

        for z in range(ZROWS // ZCH):
            pltpu.sync_copy(zbuf, acc.at[pl.ds(tid * ZROWS + z * ZCH, ZCH)])
        plsc.subcore_barrier()

        for r, (sh, dh) in enumerate(((src0, dst0), (src1, dst1),
                                      (src2, dst2))):
            off = r * (NFC * N_NODES) + fc
            pltpu.sync_copy(sh.at[tid], srcbuf)
            pltpu.sync_copy(dh.at[tid], dstbuf)

            def _off(j, c):
                for t in range(UN // 16):
                    v = srcbuf[j, pl.ds(t * 16, 16)]
                    srcbuf[j, pl.ds(t * 16, 16)] = v * NFC + off
                return c
            lax.fori_loop(0, UNITS, _off, 0)

            for b in range(K):
                pltpu.async_copy(h_hbm.at[srcbuf.at[b]],
                                 gbuf.at[b], semg.at[b])

            def _grp(g, c):
                for b in range(K):
                    pltpu.make_async_copy(h_hbm.at[srcbuf.at[0]],
                                          gbuf.at[b], semg.at[b]).wait()
                    pltpu.async_copy(gbuf.at[b],
                                     acc.at[dstbuf.at[g * K + b]],
                                     sems.at[b], add=True)

                @pl.when(g < NG - 1)
                def _next():
                    for b in range(K):
                        pltpu.make_async_copy(gbuf.at[b], acc.at[dstbuf.at[0]],
                                              sems.at[b]).wait()
                        pltpu.async_copy(h_hbm.at[srcbuf.at[(g + 1) * K + b]],
                                         gbuf.at[b], semg.at[b])
                return c
            lax.fori_loop(0, NG, _grp, 0)
            for b in range(K):
                pltpu.make_async_copy(gbuf.at[b], acc.at[dstbuf.at[0]],
                                      sems.at[b]).wait()

        plsc.subcore_barrier()

        for dch in range(DROWS // DCH):
            row0 = tid * DROWS + dch * DCH
            pltpu.sync_copy(acc.at[pl.ds(row0, DCH)], dbuf)

            def _relu(i, c):
                v = jnp.maximum(dbuf[i, pl.ds(0, FC)],
                                jnp.zeros((FC,), jnp.bfloat16))
                lo, hi = plsc.unpack(v, format=plsc.PackFormat.INTERLEAVED)
                dbuf32[i, pl.ds(0, FC // 2)] = lo
                dbuf32[i, pl.ds(FC // 2, FC // 2)] = hi
                return c
            lax.fori_loop(0, DCH, _relu, 0)
            pltpu.sync_copy(dbuf32,
                            out_hbm.at[pl.ds(row0, DCH), pl.ds(fc * FC, FC)])
        plsc.subcore_barrier()


def kernel(x, edge_index_r0, edge_index_r1, edge_index_r2,
           W_r0, W_r1, W_r2):
    h = _matmul(x, jnp.stack((W_r0, W_r1, W_r2))[:, :, jnp.array(_PERM)])
    h_all = h.reshape(NREL * N_NODES * NFC, FC)
    args = [h_all]
    for ei in (edge_index_r0, edge_index_r1, edge_index_r2):
        src = jnp.concatenate((ei[0], jnp.zeros((EPAD - E,), jnp.int32)))
        dst = jnp.concatenate((ei[1], jnp.full((EPAD - E,), DUMMY,
                                               jnp.int32)))
        args.append(src.reshape(NS, UNITS, UN))
        args.append(dst.reshape(NS, UNITS, UN))
    return _sc_agg(*args)


# speedup vs baseline: 1.2973x; 1.0184x over previous
import functools

import jax
import jax.numpy as jnp
from jax import lax
from jax.experimental import pallas as pl
from jax.experimental.pallas import tpu as pltpu
from jax.experimental.pallas import tpu_sc as plsc

N_NODES = 50000
D = 128
E = 200000
NREL = 3

NC = 2
NS = 16
FC = 32
NFC = D // FC
PASSES = NFC // NC
UN = 256
UNITS = 50
EPAD = NS * UNITS * UN

ACC_ROWS = 50176
ZROWS = ACC_ROWS // NS
ZCH = 112
EPT = EPAD // NS
K = 10
NG = UNITS // K
DUMMY = N_NODES
DROWS = N_NODES // NS
DCH = 125

_PERM = []
for _c in range(NFC):
    for _k in range(FC // 2):
        _PERM += [_c * FC + _k, _c * FC + FC // 2 + _k]


def _mm_body(x_ref, w_ref, o_ref):
    for r in range(NREL):
        o_ref[r] = jnp.dot(x_ref[...], w_ref[r],
                           preferred_element_type=jnp.float32
                           ).astype(jnp.bfloat16)


def _matmul(x, w_stack):
    mblk = 400
    return pl.pallas_call(
        _mm_body,
        grid=(N_NODES // mblk,),
        in_specs=[pl.BlockSpec((mblk, D), lambda i: (i, 0)),
                  pl.BlockSpec((NREL, D, D), lambda i: (0, 0, 0))],
        out_specs=pl.BlockSpec((NREL, mblk, D), lambda i: (0, i, 0)),
        out_shape=jax.ShapeDtypeStruct((NREL, N_NODES, D), jnp.bfloat16),
    )(x, w_stack)


_MESH = plsc.VectorSubcoreMesh(core_axis_name="c", subcore_axis_name="s")


@functools.partial(
    pl.kernel,
    out_type=jax.ShapeDtypeStruct((N_NODES, D), jnp.float32),
    mesh=_MESH,
    scratch_types=[
        pltpu.VMEM_SHARED((ACC_ROWS, FC), jnp.bfloat16),
        pltpu.VMEM((EPT,), jnp.int32),
        pltpu.VMEM((EPT,), jnp.int32),
        pltpu.VMEM((K, UN, FC), jnp.bfloat16),
        pltpu.VMEM((ZCH, FC), jnp.bfloat16),
        pltpu.VMEM((DCH, FC), jnp.bfloat16),
        pltpu.VMEM((DCH, FC), jnp.float32),
        pltpu.SemaphoreType.DMA((K,)),
        pltpu.SemaphoreType.DMA((K,)),
    ],
    compiler_params=pltpu.CompilerParams(use_tc_tiling_on_sc=False,
                                         needs_layout_passes=False),
)
def _sc_agg(h_hbm, src0, dst0, src1, dst1, src2, dst2, out_hbm,
            acc, srcbuf, dstbuf, gbuf, zbuf, dbuf, dbuf32, semg, sems):
    core = lax.axis_index("c")
    tid = lax.axis_index("s")

    def _zb(i, c):
        zbuf[i, pl.ds(0, FC)] = jnp.zeros((FC,), jnp.bfloat16)
        return c
    lax.fori_loop(0, ZCH, _zb, 0)

    for p in range(PASSES):
        fc = core * PASSES + p

        for z in range(ZROWS // ZCH):
            pltpu.sync_copy(zbuf, acc.at[pl.ds(tid * ZROWS + z * ZCH, ZCH)])
        plsc.subcore_barrier()

        for r, (sh, dh) in enumerate(((src0, dst0), (src1, dst1),
                                      (src2, dst2))):
            off = r * (NFC * N_NODES) + fc
            pltpu.sync_copy(sh.at[pl.ds(tid * EPT, EPT)], srcbuf)
            pltpu.sync_copy(dh.at[pl.ds(tid * EPT, EPT)], dstbuf)

            def _off(j, c):
                for t in range(UN // 16):
                    v = srcbuf[pl.ds(j * UN + t * 16, 16)]
                    srcbuf[pl.ds(j * UN + t * 16, 16)] = v * NFC + off
                return c
            lax.fori_loop(0, UNITS, _off, 0)

            for b in range(K):
                pltpu.async_copy(h_hbm.at[srcbuf.at[pl.ds(b * UN, UN)]],
                                 gbuf.at[b], semg.at[b])

            def _grp(g, c):
                for b in range(K):
                    pltpu.make_async_copy(h_hbm.at[srcbuf.at[pl.ds(0, UN)]],
                                          gbuf.at[b], semg.at[b]).wait()
                    pltpu.async_copy(
                        gbuf.at[b],
                        acc.at[dstbuf.at[pl.ds((g * K + b) * UN, UN)]],
                        sems.at[b], add=True)

                @pl.when(g < NG - 1)
                def _next():
                    for b in range(K):
                        pltpu.make_async_copy(
                            gbuf.at[b], acc.at[dstbuf.at[pl.ds(0, UN)]],
                            sems.at[b]).wait()
                        pltpu.async_copy(
                            h_hbm.at[
                                srcbuf.at[pl.ds(((g + 1) * K + b) * UN, UN)]],
                            gbuf.at[b], semg.at[b])
                return c
            lax.fori_loop(0, NG, _grp, 0)
            for b in range(K):
                pltpu.make_async_copy(gbuf.at[b],
                                      acc.at[dstbuf.at[pl.ds(0, UN)]],
                                      sems.at[b]).wait()

        plsc.subcore_barrier()

        for dch in range(DROWS // DCH):
            row0 = tid * DROWS + dch * DCH
            pltpu.sync_copy(acc.at[pl.ds(row0, DCH)], dbuf)

            def _relu(i, c):
                v = jnp.maximum(dbuf[i, pl.ds(0, FC)],
                                jnp.zeros((FC,), jnp.bfloat16))
                lo, hi = plsc.unpack(v, format=plsc.PackFormat.INTERLEAVED)
                dbuf32[i, pl.ds(0, FC // 2)] = lo
                dbuf32[i, pl.ds(FC // 2, FC // 2)] = hi
                return c
            lax.fori_loop(0, DCH, _relu, 0)
            pltpu.sync_copy(dbuf32,
                            out_hbm.at[pl.ds(row0, DCH), pl.ds(fc * FC, FC)])
        plsc.subcore_barrier()


def kernel(x, edge_index_r0, edge_index_r1, edge_index_r2,
           W_r0, W_r1, W_r2):
    h = _matmul(x, jnp.stack((W_r0, W_r1, W_r2))[:, :, jnp.array(_PERM)])
    h_all = h.reshape(NREL * N_NODES * NFC, FC)
    args = [h_all]
    for ei in (edge_index_r0, edge_index_r1, edge_index_r2):
        src = jnp.concatenate((ei[0], jnp.zeros((EPAD - E,), jnp.int32)))
        dst = jnp.concatenate((ei[1], jnp.full((EPAD - E,), DUMMY,
                                               jnp.int32)))
        args.append(src)
        args.append(dst)
    return _sc_agg(*args)


# speedup vs baseline: 1.3197x; 1.0173x over previous
import functools

import jax
import jax.numpy as jnp
from jax import lax
from jax.experimental import pallas as pl
from jax.experimental.pallas import tpu as pltpu
from jax.experimental.pallas import tpu_sc as plsc

N_NODES = 50000
D = 128
E = 200000
NREL = 3

NC = 2
NS = 16
FC = 32
NFC = D // FC
PASSES = NFC // NC
UN = 256
UNITS = 50
EPAD = NS * UNITS * UN

ACC_ROWS = 50176
ZROWS = ACC_ROWS // NS
ZCH = 112
EPT = EPAD // NS
K = 10
NG = UNITS // K
DUMMY = N_NODES
DROWS = N_NODES // NS
DCH = 125

_PERM = []
for _c in range(NFC):
    for _k in range(FC // 2):
        _PERM += [_c * FC + _k, _c * FC + FC // 2 + _k]


def _mm_body(x_ref, w_ref, o_ref):
    for r in range(NREL):
        o_ref[r] = jnp.dot(x_ref[...], w_ref[r],
                           preferred_element_type=jnp.float32
                           ).astype(jnp.bfloat16)


def _matmul(x, w_stack):
    mblk = 400
    return pl.pallas_call(
        _mm_body,
        grid=(N_NODES // mblk,),
        in_specs=[pl.BlockSpec((mblk, D), lambda i: (i, 0)),
                  pl.BlockSpec((NREL, D, D), lambda i: (0, 0, 0))],
        out_specs=pl.BlockSpec((NREL, mblk, D), lambda i: (0, i, 0)),
        out_shape=jax.ShapeDtypeStruct((NREL, N_NODES, D), jnp.bfloat16),
    )(x, w_stack)


_MESH = plsc.VectorSubcoreMesh(core_axis_name="c", subcore_axis_name="s")


@functools.partial(
    pl.kernel,
    out_type=jax.ShapeDtypeStruct((N_NODES, D), jnp.float32),
    mesh=_MESH,
    scratch_types=[
        pltpu.VMEM_SHARED((ACC_ROWS, FC), jnp.bfloat16),
        pltpu.VMEM((EPT,), jnp.int32),
        pltpu.VMEM((EPT,), jnp.int32),
        pltpu.VMEM((K, UN, FC), jnp.bfloat16),
        pltpu.VMEM((ZCH, FC), jnp.bfloat16),
        pltpu.VMEM((DCH, FC), jnp.bfloat16),
        pltpu.VMEM((2, DCH, FC), jnp.float32),
        pltpu.SemaphoreType.DMA((K,)),
        pltpu.SemaphoreType.DMA((K,)),
    ],
    compiler_params=pltpu.CompilerParams(use_tc_tiling_on_sc=False,
                                         needs_layout_passes=False),
)
def _sc_agg(h_hbm, src0, dst0, src1, dst1, src2, dst2, out_hbm,
            acc, srcbuf, dstbuf, gbuf, zbuf, dbuf, dbuf32, semg, sems):
    core = lax.axis_index("c")
    tid = lax.axis_index("s")

    def _zb(i, c):
        zbuf[i, pl.ds(0, FC)] = jnp.zeros((FC,), jnp.bfloat16)
        return c
    lax.fori_loop(0, ZCH, _zb, 0)

    for p in range(PASSES):
        fc = core * PASSES + p

        for z in range(ZROWS // ZCH):
            pltpu.async_copy(zbuf, acc.at[pl.ds(tid * ZROWS + z * ZCH, ZCH)],
                             semg.at[0])
        for z in range(ZROWS // ZCH):
            pltpu.make_async_copy(
                zbuf, acc.at[pl.ds(tid * ZROWS, ZCH)], semg.at[0]).wait()
        plsc.subcore_barrier()

        for r, (sh, dh) in enumerate(((src0, dst0), (src1, dst1),
                                      (src2, dst2))):
            off = r * (NFC * N_NODES) + fc
            pltpu.sync_copy(sh.at[pl.ds(tid * EPT, EPT)], srcbuf)
            pltpu.sync_copy(dh.at[pl.ds(tid * EPT, EPT)], dstbuf)

            def _off(j, c):
                for t in range(UN // 16):
                    v = srcbuf[pl.ds(j * UN + t * 16, 16)]
                    srcbuf[pl.ds(j * UN + t * 16, 16)] = v * NFC + off
                return c
            lax.fori_loop(0, UNITS, _off, 0)

            for b in range(K):
                pltpu.async_copy(h_hbm.at[srcbuf.at[pl.ds(b * UN, UN)]],
                                 gbuf.at[b], semg.at[b])

            def _grp(g, c):
                for b in range(K):
                    pltpu.make_async_copy(h_hbm.at[srcbuf.at[pl.ds(0, UN)]],
                                          gbuf.at[b], semg.at[b]).wait()
                    pltpu.async_copy(
                        gbuf.at[b],
                        acc.at[dstbuf.at[pl.ds((g * K + b) * UN, UN)]],
                        sems.at[b], add=True)

                @pl.when(g < NG - 1)
                def _next():
                    for b in range(K):
                        pltpu.make_async_copy(
                            gbuf.at[b], acc.at[dstbuf.at[pl.ds(0, UN)]],
                            sems.at[b]).wait()
                        pltpu.async_copy(
                            h_hbm.at[
                                srcbuf.at[pl.ds(((g + 1) * K + b) * UN, UN)]],
                            gbuf.at[b], semg.at[b])
                return c
            lax.fori_loop(0, NG, _grp, 0)
            for b in range(K):
                pltpu.make_async_copy(gbuf.at[b],
                                      acc.at[dstbuf.at[pl.ds(0, UN)]],
                                      sems.at[b]).wait()

        plsc.subcore_barrier()

        for dch in range(DROWS // DCH):
            row0 = tid * DROWS + dch * DCH
            par = dch % 2
            pltpu.sync_copy(acc.at[pl.ds(row0, DCH)], dbuf)
            if dch >= 2:
                pltpu.make_async_copy(
                    dbuf32.at[par],
                    out_hbm.at[pl.ds(tid * DROWS, DCH), pl.ds(fc * FC, FC)],
                    sems.at[par]).wait()

            def _relu(i, c):
                v = jnp.maximum(dbuf[i, pl.ds(0, FC)],
                                jnp.zeros((FC,), jnp.bfloat16))
                lo, hi = plsc.unpack(v, format=plsc.PackFormat.INTERLEAVED)
                dbuf32[par, i, pl.ds(0, FC // 2)] = lo
                dbuf32[par, i, pl.ds(FC // 2, FC // 2)] = hi
                return c
            lax.fori_loop(0, DCH, _relu, 0)
            pltpu.async_copy(dbuf32.at[par],
                             out_hbm.at[pl.ds(row0, DCH), pl.ds(fc * FC, FC)],
                             sems.at[par])
        for par in range(2):
            pltpu.make_async_copy(
                dbuf32.at[par],
                out_hbm.at[pl.ds(tid * DROWS, DCH), pl.ds(fc * FC, FC)],
                sems.at[par]).wait()
        plsc.subcore_barrier()


def kernel(x, edge_index_r0, edge_index_r1, edge_index_r2,
           W_r0, W_r1, W_r2):
    h = _matmul(x, jnp.stack((W_r0, W_r1, W_r2))[:, :, jnp.array(_PERM)])
    h_all = h.reshape(NREL * N_NODES * NFC, FC)
    args = [h_all]
    for ei in (edge_index_r0, edge_index_r1, edge_index_r2):
        src = jnp.concatenate((ei[0], jnp.zeros((EPAD - E,), jnp.int32)))
        dst = jnp.concatenate((ei[1], jnp.full((EPAD - E,), DUMMY,
                                               jnp.int32)))
        args.append(src)
        args.append(dst)
    return _sc_agg(*args)
